# baseline (speedup 1.0000x reference)
import math

import jax
import jax.numpy as jnp
from jax.experimental import pallas as pl
from jax.experimental.pallas import tpu as pltpu

V = 32000
SMOOTH_A = 0.1 / (V - 1)
CONF_C = 0.9
K_CONST = (V - 1) * SMOOTH_A * math.log(SMOOTH_A) + CONF_C * math.log(CONF_C)

R_BLK = 2048
V_BLK = 3200
N_ROWS = 4096
NR = N_ROWS // R_BLK
NV = V // V_BLK


def _loss_body(tgt_ref, pred_ref, out_ref, acc_ref, cnt_ref):
    i = pl.program_id(0)
    j = pl.program_id(1)

    @pl.when((i == 0) & (j == 0))
    def _init():
        acc_ref[0] = 0.0
        cnt_ref[0] = 0.0

    tgt = tgt_ref[0, 0, :]
    maskf = (tgt > 0).astype(jnp.float32)

    @pl.when(j == 0)
    def _count():
        cnt_ref[0] += jnp.sum(maskf)

    pred = pred_ref[...]
    col = jax.lax.broadcasted_iota(jnp.int32, (R_BLK, V_BLK), 1) + j * V_BLK
    w = jnp.where(col == tgt[:, None], CONF_C, SMOOTH_A)
    row_part = jnp.sum(pred * w, axis=1)
    acc_ref[0] += jnp.sum(row_part * maskf)

    @pl.when((i == NR - 1) & (j == NV - 1))
    def _fin():
        out_ref[0] = K_CONST - acc_ref[0] / cnt_ref[0]


def kernel(prediction, target):
    pred = prediction.reshape(N_ROWS, V)
    tgt = target.reshape(NR, 1, R_BLK).astype(jnp.int32)
    out = pl.pallas_call(
        _loss_body,
        grid=(NR, NV),
        in_specs=[
            pl.BlockSpec((1, 1, R_BLK), lambda i, j: (i, 0, 0)),
            pl.BlockSpec((R_BLK, V_BLK), lambda i, j: (i, j)),
        ],
        out_specs=pl.BlockSpec(memory_space=pltpu.SMEM),
        out_shape=jax.ShapeDtypeStruct((1,), jnp.float32),
        scratch_shapes=[
            pltpu.SMEM((1,), jnp.float32),
            pltpu.SMEM((1,), jnp.float32),
        ],
    )(tgt, pred)
    return out[0]
